# baseline (speedup 1.0000x reference)
import jax
import jax.numpy as jnp
from jax.experimental import pallas as pl

_B, _L, _NA, _EMB, _HID = 64, 128, 16, 256, 64
_N = _B * _L
_NS = 1 + _NA


def _body(fwd_ref, bwd_ref, rew_ref, pif_ref, ifl_ref, w1_ref, b1_ref,
          w2_ref, b2_ref, out_ref):
    j = pl.program_id(0)

    def head(x):
        h = jnp.tanh(
            jnp.dot(x, w1_ref[...], preferred_element_type=jnp.float32)
            + b1_ref[...])
        return (jnp.dot(h, w2_ref[...], preferred_element_type=jnp.float32)
                + b2_ref[...])

    @pl.when(j == 0)
    def _():
        y = head(fwd_ref[...])
        fout = jnp.sum(jax.nn.softplus(y), axis=1)
        out_ref[0, :] = jnp.zeros((_N,), jnp.float32)
        out_ref[1, :] = fout
        out_ref[2:3, :] = rew_ref[...]
        out_ref[3:4, :] = pif_ref[...] * jnp.exp(ifl_ref[...])

    @pl.when(j > 0)
    def _():
        y = head(bwd_ref[...])
        lane = jax.lax.broadcasted_iota(jnp.int32, y.shape, 1)
        z = jnp.sum(jnp.where(lane == j - 1, y, 0.0), axis=1)
        out_ref[0, :] = out_ref[0, :] + jax.nn.softplus(z)


def kernel(forward_edges, backward_edges, paths_reward, path_init_flow,
           initial_flow, W1, b1, W2, b2):
    fwd = forward_edges.reshape(_N, _NS * _EMB)
    bwd = backward_edges.reshape(_N, _NS * _EMB)
    rew = paths_reward.reshape(1, _N)
    pif = path_init_flow.reshape(1, _N)
    ifl = initial_flow.reshape(1, 1)
    b1r = b1.reshape(1, _HID)
    b2r = b2.reshape(1, _NA)

    out = pl.pallas_call(
        _body,
        grid=(_NS,),
        in_specs=[
            pl.BlockSpec((_N, _EMB), lambda j: (0, 0)),
            pl.BlockSpec((_N, _EMB), lambda j: (0, jnp.maximum(j, 1))),
            pl.BlockSpec((1, _N), lambda j: (0, 0)),
            pl.BlockSpec((1, _N), lambda j: (0, 0)),
            pl.BlockSpec((1, 1), lambda j: (0, 0)),
            pl.BlockSpec((_EMB, _HID), lambda j: (0, 0)),
            pl.BlockSpec((1, _HID), lambda j: (0, 0)),
            pl.BlockSpec((_HID, _NA), lambda j: (0, 0)),
            pl.BlockSpec((1, _NA), lambda j: (0, 0)),
        ],
        out_specs=pl.BlockSpec((4, _N), lambda j: (0, 0)),
        out_shape=jax.ShapeDtypeStruct((4, _N), jnp.float32),
    )(fwd, bwd, rew, pif, ifl, W1, b1r, W2, b2r)

    return out.T.reshape(_B, _L, 4)


# speedup vs baseline: 1.3268x; 1.3268x over previous
import jax
import jax.numpy as jnp
from jax.experimental import pallas as pl
from jax.experimental.pallas import tpu as pltpu

_B, _L, _NA, _EMB, _HID = 64, 128, 16, 256, 64
_N = _B * _L
_NS = 1 + _NA


def _body(fwd_ref, bwd_ref, rew_ref, pif_ref, ifl_ref, w1_ref, b1_ref,
          w2_ref, b2_ref, ones_ref, out_ref, yf_scr, z_scr):
    j = pl.program_id(0)

    def head(x):
        h = jnp.tanh(
            jnp.dot(x, w1_ref[...], preferred_element_type=jnp.float32)
            + b1_ref[...])
        return (jnp.dot(h, w2_ref[...], preferred_element_type=jnp.float32)
                + b2_ref[...])

    @pl.when(j == 0)
    def _():
        yf_scr[...] = head(fwd_ref[...])

    @pl.when(j > 0)
    def _():
        y = head(bwd_ref[...])
        lane = jax.lax.broadcasted_iota(jnp.int32, (_N, _NA), 1)
        z_scr[...] = jnp.where(lane == j - 1, y, z_scr[...])

    @pl.when(j == _NS - 1)
    def _():
        fin = jnp.dot(jax.nn.softplus(z_scr[...]), ones_ref[...],
                      preferred_element_type=jnp.float32)
        fout = jnp.dot(jax.nn.softplus(yf_scr[...]), ones_ref[...],
                       preferred_element_type=jnp.float32)
        out_ref[:, 0:1] = fin
        out_ref[:, 1:2] = fout
        out_ref[:, 2:3] = rew_ref[...]
        out_ref[:, 3:4] = pif_ref[...] * jnp.exp(ifl_ref[...])


def kernel(forward_edges, backward_edges, paths_reward, path_init_flow,
           initial_flow, W1, b1, W2, b2):
    fwd = forward_edges.reshape(_N, _NS * _EMB)
    bwd = backward_edges.reshape(_N, _NS * _EMB)
    rew = paths_reward.reshape(_N, 1)
    pif = path_init_flow.reshape(_N, 1)
    ifl = initial_flow.reshape(1, 1)
    b1r = b1.reshape(1, _HID)
    b2r = b2.reshape(1, _NA)
    ones = jnp.ones((_NA, 1), jnp.float32)

    out = pl.pallas_call(
        _body,
        grid=(_NS,),
        in_specs=[
            pl.BlockSpec((_N, _EMB), lambda j: (0, 0)),
            pl.BlockSpec((_N, _EMB), lambda j: (0, jnp.maximum(j, 1))),
            pl.BlockSpec((_N, 1), lambda j: (0, 0)),
            pl.BlockSpec((_N, 1), lambda j: (0, 0)),
            pl.BlockSpec((1, 1), lambda j: (0, 0)),
            pl.BlockSpec((_EMB, _HID), lambda j: (0, 0)),
            pl.BlockSpec((1, _HID), lambda j: (0, 0)),
            pl.BlockSpec((_HID, _NA), lambda j: (0, 0)),
            pl.BlockSpec((1, _NA), lambda j: (0, 0)),
            pl.BlockSpec((_NA, 1), lambda j: (0, 0)),
        ],
        out_specs=pl.BlockSpec((_N, 4), lambda j: (0, 0)),
        out_shape=jax.ShapeDtypeStruct((_N, 4), jnp.float32),
        scratch_shapes=[
            pltpu.VMEM((_N, _NA), jnp.float32),
            pltpu.VMEM((_N, _NA), jnp.float32),
        ],
    )(fwd, bwd, rew, pif, ifl, W1, b1r, W2, b2r, ones)

    return out.reshape(_B, _L, 4)


# speedup vs baseline: 3.0931x; 2.3311x over previous
import jax
import jax.numpy as jnp
from jax.experimental import pallas as pl
from jax.experimental.pallas import tpu as pltpu

_B, _L, _NA, _EMB, _HID = 64, 128, 16, 256, 64
_N = _B * _L
_NS = 1 + _NA
_R = 1024


def _body(fwd_ref, bwd_ref, rew_ref, pif_ref, ifl_ref, w1_ref, b1_ref,
          w2_ref, b2_ref, ones_ref, out_ref):
    def head(x):
        h = jnp.tanh(
            jnp.dot(x, w1_ref[...], preferred_element_type=jnp.float32)
            + b1_ref[...])
        return (jnp.dot(h, w2_ref[...], preferred_element_type=jnp.float32)
                + b2_ref[...])

    yf = head(fwd_ref[...])
    lane = jax.lax.broadcasted_iota(jnp.int32, (_R, _NA), 1)
    z = jnp.zeros((_R, _NA), jnp.float32)
    for s in range(1, _NS):
        ys = head(bwd_ref[:, s, :])
        z = jnp.where(lane == s - 1, ys, z)

    fin = jnp.dot(jax.nn.softplus(z), ones_ref[...],
                  preferred_element_type=jnp.float32)
    fout = jnp.dot(jax.nn.softplus(yf), ones_ref[...],
                   preferred_element_type=jnp.float32)
    out_ref[:, 0:1] = fin
    out_ref[:, 1:2] = fout
    out_ref[:, 2:3] = rew_ref[...]
    out_ref[:, 3:4] = pif_ref[...] * jnp.exp(ifl_ref[...])


def kernel(forward_edges, backward_edges, paths_reward, path_init_flow,
           initial_flow, W1, b1, W2, b2):
    fwd = forward_edges[:, :, 0].reshape(_N, _EMB)
    bwd = backward_edges.reshape(_N, _NS, _EMB)
    rew = paths_reward.reshape(_N, 1)
    pif = path_init_flow.reshape(_N, 1)
    ifl = initial_flow.reshape(1, 1)
    b1r = b1.reshape(1, _HID)
    b2r = b2.reshape(1, _NA)
    ones = jnp.ones((_NA, 1), jnp.float32)

    out = pl.pallas_call(
        _body,
        grid=(_N // _R,),
        in_specs=[
            pl.BlockSpec((_R, _EMB), lambda i: (i, 0)),
            pl.BlockSpec((_R, _NS, _EMB), lambda i: (i, 0, 0)),
            pl.BlockSpec((_R, 1), lambda i: (i, 0)),
            pl.BlockSpec((_R, 1), lambda i: (i, 0)),
            pl.BlockSpec((1, 1), lambda i: (0, 0)),
            pl.BlockSpec((_EMB, _HID), lambda i: (0, 0)),
            pl.BlockSpec((1, _HID), lambda i: (0, 0)),
            pl.BlockSpec((_HID, _NA), lambda i: (0, 0)),
            pl.BlockSpec((1, _NA), lambda i: (0, 0)),
            pl.BlockSpec((_NA, 1), lambda i: (0, 0)),
        ],
        out_specs=pl.BlockSpec((_R, 4), lambda i: (i, 0)),
        out_shape=jax.ShapeDtypeStruct((_N, 4), jnp.float32),
        compiler_params=pltpu.CompilerParams(
            dimension_semantics=("parallel",)),
    )(fwd, bwd, rew, pif, ifl, W1, b1r, W2, b2r, ones)

    return out.reshape(_B, _L, 4)
